# baseline (device time: 153781 ns/iter reference)
import jax
import jax.numpy as jnp
from jax import lax
from jax.experimental import pallas as pl
from jax.experimental.pallas import tpu as pltpu

N_Z = 4
CHUNK = 1024
D = 1024


def kernel(partial, gamma):
    gamma2 = gamma.reshape(1, D)

    def body(x_ref, g_ref, out_ref, comm_ref, send_sems, recv_sems):
        my_x = lax.axis_index("x")
        my_y = lax.axis_index("y")
        my_z = lax.axis_index("z")
        left = (my_z + N_Z - 1) % N_Z
        right = (my_z + 1) % N_Z

        barrier = pltpu.get_barrier_semaphore()
        for nbr in (left, right):
            pl.semaphore_signal(
                barrier, inc=1,
                device_id=(my_x, my_y, nbr),
                device_id_type=pl.DeviceIdType.MESH,
            )
        pl.semaphore_wait(barrier, 2)

        def local_chunk(c):
            return x_ref[0, pl.ds(c * CHUNK, CHUNK), :]

        for s in range(N_Z - 1):
            if s == 0:
                src = x_ref.at[0, pl.ds(left * CHUNK, CHUNK), :]
            else:
                src = comm_ref.at[s - 1]
            rdma = pltpu.make_async_remote_copy(
                src_ref=src,
                dst_ref=comm_ref.at[s],
                send_sem=send_sems.at[s],
                recv_sem=recv_sems.at[s],
                device_id=(my_x, my_y, right),
                device_id_type=pl.DeviceIdType.MESH,
            )
            rdma.start()
            rdma.wait()
            c = (my_z + 2 * N_Z - s - 2) % N_Z
            comm_ref[s, :, :] = comm_ref[s, :, :] + local_chunk(c)

        y = comm_ref[N_Z - 2, :, :]
        ms = jnp.mean(y * y, axis=-1, keepdims=True) + 1e-6
        out_ref[:, :] = y * lax.rsqrt(ms) * g_ref[0, :]

    return pl.pallas_call(
        body,
        out_shape=jax.ShapeDtypeStruct((CHUNK, D), jnp.float32),
        in_specs=[
            pl.BlockSpec(memory_space=pltpu.VMEM),
            pl.BlockSpec(memory_space=pltpu.VMEM),
        ],
        out_specs=pl.BlockSpec(memory_space=pltpu.VMEM),
        scratch_shapes=[
            pltpu.VMEM((N_Z - 1, CHUNK, D), jnp.float32),
            pltpu.SemaphoreType.DMA((N_Z - 1,)),
            pltpu.SemaphoreType.DMA((N_Z - 1,)),
        ],
        compiler_params=pltpu.CompilerParams(collective_id=0),
    )(partial, gamma2)


# device time: 66930 ns/iter; 2.2976x vs baseline; 2.2976x over previous
import functools

import jax
import jax.numpy as jnp
from jax import lax
from jax.experimental import pallas as pl
from jax.experimental.pallas import tpu as pltpu

N_Z = 4
N_XY = 8
CHUNK = 1024
D = 1024
SL = D // N_XY


def kernel(partial, gamma):
    gamma2 = gamma.reshape(1, D)

    def body(x_ref, g_ref, out_ref, ph1_ref, y_ref,
             p1_send, p1_recv, cw_send, cw_recv, ccw_send, ccw_recv):
        my_x = lax.axis_index("x")
        my_y = lax.axis_index("y")
        my_z = lax.axis_index("z")
        zl = (my_z + N_Z - 1) % N_Z
        zr = (my_z + 1) % N_Z

        p = jnp.where(my_x == 0, my_y, 7 - my_y)

        def ring_coords(q):
            q = q % N_XY
            return q // 4, jnp.where(q < 4, q, 7 - q)

        cw_x, cw_y = ring_coords(p + 1)
        ccw_x, ccw_y = ring_coords(p + N_XY - 1)

        barrier = pltpu.get_barrier_semaphore()
        for nz in (zl, zr):
            pl.semaphore_signal(
                barrier, inc=1,
                device_id=(my_x, my_y, nz),
                device_id_type=pl.DeviceIdType.MESH,
            )
        pl.semaphore_wait(barrier, 2)

        col0 = p * SL

        def local_chunk(c):
            return x_ref[0, pl.ds(c * CHUNK, CHUNK), pl.ds(col0, SL)]

        for s in range(N_Z - 1):
            if s == 0:
                src = x_ref.at[0, pl.ds(zl * CHUNK, CHUNK), pl.ds(col0, SL)]
            else:
                src = ph1_ref.at[s - 1]
            rdma = pltpu.make_async_remote_copy(
                src_ref=src,
                dst_ref=ph1_ref.at[s],
                send_sem=p1_send.at[s],
                recv_sem=p1_recv.at[s],
                device_id=(my_x, my_y, zr),
                device_id_type=pl.DeviceIdType.MESH,
            )
            rdma.start()
            rdma.wait()
            c = (my_z + 2 * N_Z - s - 2) % N_Z
            if s < N_Z - 2:
                ph1_ref[s, :, :] = ph1_ref[s, :, :] + local_chunk(c)
            else:
                y_ref[:, pl.ds(col0, SL)] = ph1_ref[s, :, :] + local_chunk(c)

        @functools.partial(pl.run_scoped, xy_bar=pltpu.SemaphoreType.REGULAR)
        def _(xy_bar):
            for qx, qy in ((cw_x, cw_y), (ccw_x, ccw_y)):
                pl.semaphore_signal(
                    xy_bar, inc=1,
                    device_id=(qx, qy, my_z),
                    device_id_type=pl.DeviceIdType.MESH,
                )
            pl.semaphore_wait(xy_bar, 2)

        for h in range(N_XY // 2):
            s_cw = (p + N_XY - h) % N_XY
            cw = pltpu.make_async_remote_copy(
                src_ref=y_ref.at[:, pl.ds(s_cw * SL, SL)],
                dst_ref=y_ref.at[:, pl.ds(s_cw * SL, SL)],
                send_sem=cw_send.at[h],
                recv_sem=cw_recv.at[h],
                device_id=(cw_x, cw_y, my_z),
                device_id_type=pl.DeviceIdType.MESH,
            )
            cw.start()
            if h < N_XY // 2 - 1:
                s_ccw = (p + h) % N_XY
                ccw = pltpu.make_async_remote_copy(
                    src_ref=y_ref.at[:, pl.ds(s_ccw * SL, SL)],
                    dst_ref=y_ref.at[:, pl.ds(s_ccw * SL, SL)],
                    send_sem=ccw_send.at[h],
                    recv_sem=ccw_recv.at[h],
                    device_id=(ccw_x, ccw_y, my_z),
                    device_id_type=pl.DeviceIdType.MESH,
                )
                ccw.start()
                ccw.wait()
            cw.wait()

        y = y_ref[:, :]
        ms = jnp.mean(y * y, axis=-1, keepdims=True) + 1e-6
        out_ref[:, :] = y * lax.rsqrt(ms) * g_ref[0, :]

    return pl.pallas_call(
        body,
        out_shape=jax.ShapeDtypeStruct((CHUNK, D), jnp.float32),
        in_specs=[
            pl.BlockSpec(memory_space=pltpu.VMEM),
            pl.BlockSpec(memory_space=pltpu.VMEM),
        ],
        out_specs=pl.BlockSpec(memory_space=pltpu.VMEM),
        scratch_shapes=[
            pltpu.VMEM((N_Z - 1, CHUNK, SL), jnp.float32),
            pltpu.VMEM((CHUNK, D), jnp.float32),
            pltpu.SemaphoreType.DMA((N_Z - 1,)),
            pltpu.SemaphoreType.DMA((N_Z - 1,)),
            pltpu.SemaphoreType.DMA((N_XY // 2,)),
            pltpu.SemaphoreType.DMA((N_XY // 2,)),
            pltpu.SemaphoreType.DMA((N_XY // 2 - 1,)),
            pltpu.SemaphoreType.DMA((N_XY // 2 - 1,)),
        ],
        compiler_params=pltpu.CompilerParams(collective_id=0),
    )(partial, gamma2)


# device time: 64003 ns/iter; 2.4027x vs baseline; 1.0457x over previous
import jax
import jax.numpy as jnp
from jax import lax
from jax.experimental import pallas as pl
from jax.experimental.pallas import tpu as pltpu

N_Z = 4
N_XY = 8
CHUNK = 1024
D = 1024
SL = D // N_XY
HALF = CHUNK // 2


def kernel(partial, gamma):
    gamma2 = gamma.reshape(1, D)

    def body(x_ref, g_ref, out_ref, ph1_ref, y_ref,
             p1_send, p1_recv, cw_send, cw_recv, ccw_send, ccw_recv,
             xy_bar):
        my_x = lax.axis_index("x")
        my_y = lax.axis_index("y")
        my_z = lax.axis_index("z")
        zl = (my_z + N_Z - 1) % N_Z
        zr = (my_z + 1) % N_Z

        p = jnp.where(my_x == 0, my_y, 7 - my_y)

        def ring_coords(q):
            q = q % N_XY
            return q // 4, jnp.where(q < 4, q, 7 - q)

        cw_x, cw_y = ring_coords(p + 1)
        ccw_x, ccw_y = ring_coords(p + N_XY - 1)

        barrier = pltpu.get_barrier_semaphore()
        for nz in (zl, zr):
            pl.semaphore_signal(
                barrier, inc=1,
                device_id=(my_x, my_y, nz),
                device_id_type=pl.DeviceIdType.MESH,
            )
        pl.semaphore_wait(barrier, 2)

        for qx, qy in ((cw_x, cw_y), (ccw_x, ccw_y)):
            pl.semaphore_signal(
                xy_bar, inc=1,
                device_id=(qx, qy, my_z),
                device_id_type=pl.DeviceIdType.MESH,
            )

        col0 = p * SL

        def local_chunk(c):
            return x_ref[0, pl.ds(c * CHUNK, CHUNK), pl.ds(col0, SL)]

        ph1_rdmas = []
        for s in range(N_Z - 1):
            if s == 0:
                src = x_ref.at[0, pl.ds(zl * CHUNK, CHUNK), pl.ds(col0, SL)]
            else:
                src = ph1_ref.at[s - 1]
            rdma = pltpu.make_async_remote_copy(
                src_ref=src,
                dst_ref=ph1_ref.at[s],
                send_sem=p1_send.at[s],
                recv_sem=p1_recv.at[s],
                device_id=(my_x, my_y, zr),
                device_id_type=pl.DeviceIdType.MESH,
            )
            rdma.start()
            ph1_rdmas.append(rdma)
            rdma.wait_recv()
            c = (my_z + 2 * N_Z - s - 2) % N_Z
            if s < N_Z - 2:
                ph1_ref[s, :, :] = ph1_ref[s, :, :] + local_chunk(c)
            else:
                y_ref[:, pl.ds(col0, SL)] = ph1_ref[s, :, :] + local_chunk(c)
        for rdma in ph1_rdmas:
            rdma.wait_send()

        pl.semaphore_wait(xy_bar, 2)

        def slice_ref(q, rows=None):
            if rows is None:
                return y_ref.at[:, pl.ds((q % N_XY) * SL, SL)]
            return y_ref.at[pl.ds(rows, HALF), pl.ds((q % N_XY) * SL, SL)]

        def slice_val(q):
            return y_ref[:, pl.ds((q % N_XY) * SL, SL)]

        def ssq_of(q):
            v = slice_val(q)
            return jnp.sum(v * v, axis=-1, keepdims=True)

        ph2_rdmas = []
        ssq = ssq_of(p)
        for h in range(4):
            last = h == 3
            cw = pltpu.make_async_remote_copy(
                src_ref=slice_ref(p - h, rows=None if not last else 0),
                dst_ref=slice_ref(p - h, rows=None if not last else 0),
                send_sem=cw_send.at[h],
                recv_sem=cw_recv.at[h],
                device_id=(cw_x, cw_y, my_z),
                device_id_type=pl.DeviceIdType.MESH,
            )
            cw.start()
            ccw = pltpu.make_async_remote_copy(
                src_ref=slice_ref(p + h, rows=None if not last else HALF),
                dst_ref=slice_ref(p + h, rows=None if not last else HALF),
                send_sem=ccw_send.at[h],
                recv_sem=ccw_recv.at[h],
                device_id=(ccw_x, ccw_y, my_z),
                device_id_type=pl.DeviceIdType.MESH,
            )
            ccw.start()
            ph2_rdmas.append(cw)
            ph2_rdmas.append(ccw)
            if h > 0:
                ssq = ssq + ssq_of(p - h) + ssq_of(p + h)
            ccw.wait_recv()
            cw.wait_recv()
        ssq = ssq + ssq_of(p + 4)
        for rdma in ph2_rdmas:
            rdma.wait_send()

        ms = ssq * (1.0 / D) + 1e-6
        out_ref[:, :] = y_ref[:, :] * lax.rsqrt(ms) * g_ref[0, :]

    return pl.pallas_call(
        body,
        out_shape=jax.ShapeDtypeStruct((CHUNK, D), jnp.float32),
        in_specs=[
            pl.BlockSpec(memory_space=pltpu.VMEM),
            pl.BlockSpec(memory_space=pltpu.VMEM),
        ],
        out_specs=pl.BlockSpec(memory_space=pltpu.VMEM),
        scratch_shapes=[
            pltpu.VMEM((N_Z - 1, CHUNK, SL), jnp.float32),
            pltpu.VMEM((CHUNK, D), jnp.float32),
            pltpu.SemaphoreType.DMA((N_Z - 1,)),
            pltpu.SemaphoreType.DMA((N_Z - 1,)),
            pltpu.SemaphoreType.DMA((4,)),
            pltpu.SemaphoreType.DMA((4,)),
            pltpu.SemaphoreType.DMA((4,)),
            pltpu.SemaphoreType.DMA((4,)),
            pltpu.SemaphoreType.REGULAR,
        ],
        compiler_params=pltpu.CompilerParams(collective_id=0),
    )(partial, gamma2)


# device time: 56737 ns/iter; 2.7104x vs baseline; 1.1281x over previous
import jax
import jax.numpy as jnp
from jax import lax
from jax.experimental import pallas as pl
from jax.experimental.pallas import tpu as pltpu

N_Z = 4
N_XY = 8
CHUNK = 1024
D = 1024
SL = D // N_XY
HR = CHUNK // 2


def kernel(partial, gamma):
    gamma2 = gamma.reshape(1, D)

    def body(x_ref, g_ref, out_ref, ph1_ref, y_ref,
             p1_send, p1_recv, cw_send, cw_recv, ccw_send, ccw_recv,
             xy_bar):
        my_x = lax.axis_index("x")
        my_y = lax.axis_index("y")
        my_z = lax.axis_index("z")
        zl = (my_z + N_Z - 1) % N_Z
        zr = (my_z + 1) % N_Z

        p = jnp.where(my_x == 0, my_y, 7 - my_y)

        def ring_coords(q):
            q = q % N_XY
            return q // 4, jnp.where(q < 4, q, 7 - q)

        cw_x, cw_y = ring_coords(p + 1)
        ccw_x, ccw_y = ring_coords(p + N_XY - 1)

        barrier = pltpu.get_barrier_semaphore()
        for nz in (zl, zr):
            pl.semaphore_signal(
                barrier, inc=1,
                device_id=(my_x, my_y, nz),
                device_id_type=pl.DeviceIdType.MESH,
            )
        pl.semaphore_wait(barrier, 2)

        for qx, qy in ((cw_x, cw_y), (ccw_x, ccw_y)):
            pl.semaphore_signal(
                xy_bar, inc=1,
                device_id=(qx, qy, my_z),
                device_id_type=pl.DeviceIdType.MESH,
            )

        col0 = p * SL

        ph1_rd = [[None, None] for _ in range(N_Z - 1)]

        def start_ph1(s, t):
            if s == 0:
                src = x_ref.at[0, pl.ds(zl * CHUNK + t * HR, HR), pl.ds(col0, SL)]
            else:
                src = ph1_ref.at[s - 1, pl.ds(t * HR, HR), :]
            rdma = pltpu.make_async_remote_copy(
                src_ref=src,
                dst_ref=ph1_ref.at[s, pl.ds(t * HR, HR), :],
                send_sem=p1_send.at[t, s],
                recv_sem=p1_recv.at[t, s],
                device_id=(my_x, my_y, zr),
                device_id_type=pl.DeviceIdType.MESH,
            )
            rdma.start()
            ph1_rd[s][t] = rdma

        start_ph1(0, 0)
        start_ph1(0, 1)
        for s in range(N_Z - 1):
            c = (my_z + 2 * N_Z - s - 2) % N_Z
            for t in (0, 1):
                ph1_rd[s][t].wait_recv()
                local = x_ref[0, pl.ds(c * CHUNK + t * HR, HR), pl.ds(col0, SL)]
                if s < N_Z - 2:
                    ph1_ref[s, pl.ds(t * HR, HR), :] = (
                        ph1_ref[s, pl.ds(t * HR, HR), :] + local
                    )
                    start_ph1(s + 1, t)
                else:
                    y_ref[pl.ds(t * HR, HR), pl.ds(col0, SL)] = (
                        ph1_ref[s, pl.ds(t * HR, HR), :] + local
                    )

        pl.semaphore_wait(xy_bar, 2)

        def slice_ref(q, t):
            return y_ref.at[pl.ds(t * HR, HR), pl.ds((q % N_XY) * SL, SL)]

        def ssq_of(q):
            v = y_ref[:, pl.ds((q % N_XY) * SL, SL)]
            return jnp.sum(v * v, axis=-1, keepdims=True)

        streams = [
            ("cw", 0, 4, cw_send, cw_recv, (cw_x, cw_y)),
            ("cw", 1, 3, cw_send, cw_recv, (cw_x, cw_y)),
            ("ccw", 0, 3, ccw_send, ccw_recv, (ccw_x, ccw_y)),
            ("ccw", 1, 4, ccw_send, ccw_recv, (ccw_x, ccw_y)),
        ]
        ph2_rd = {}

        def start_ph2(si, h):
            d, t, _, ssem, rsem, (qx, qy) = streams[si]
            q = (p - h) if d == "cw" else (p + h)
            rdma = pltpu.make_async_remote_copy(
                src_ref=slice_ref(q, t),
                dst_ref=slice_ref(q, t),
                send_sem=ssem.at[t, h],
                recv_sem=rsem.at[t, h],
                device_id=(qx, qy, my_z),
                device_id_type=pl.DeviceIdType.MESH,
            )
            rdma.start()
            ph2_rd[(si, h)] = rdma

        for si in range(4):
            start_ph2(si, 0)
        ssq = ssq_of(p)
        for h in range(4):
            if h > 0:
                ssq = ssq + ssq_of(p - h) + ssq_of(p + h)
            for si in range(4):
                nh = streams[si][2]
                if h < nh:
                    ph2_rd[(si, h)].wait_recv()
                    if h + 1 < nh:
                        start_ph2(si, h + 1)
        ssq = ssq + ssq_of(p + 4)

        for row in ph1_rd:
            for rdma in row:
                rdma.wait_send()
        for rdma in ph2_rd.values():
            rdma.wait_send()

        ms = ssq * (1.0 / D) + 1e-6
        out_ref[:, :] = y_ref[:, :] * lax.rsqrt(ms) * g_ref[0, :]

    return pl.pallas_call(
        body,
        out_shape=jax.ShapeDtypeStruct((CHUNK, D), jnp.float32),
        in_specs=[
            pl.BlockSpec(memory_space=pltpu.VMEM),
            pl.BlockSpec(memory_space=pltpu.VMEM),
        ],
        out_specs=pl.BlockSpec(memory_space=pltpu.VMEM),
        scratch_shapes=[
            pltpu.VMEM((N_Z - 1, CHUNK, SL), jnp.float32),
            pltpu.VMEM((CHUNK, D), jnp.float32),
            pltpu.SemaphoreType.DMA((2, N_Z - 1)),
            pltpu.SemaphoreType.DMA((2, N_Z - 1)),
            pltpu.SemaphoreType.DMA((2, 4)),
            pltpu.SemaphoreType.DMA((2, 4)),
            pltpu.SemaphoreType.DMA((2, 4)),
            pltpu.SemaphoreType.DMA((2, 4)),
            pltpu.SemaphoreType.REGULAR,
        ],
        compiler_params=pltpu.CompilerParams(collective_id=0),
    )(partial, gamma2)


# device time: 53862 ns/iter; 2.8551x vs baseline; 1.0534x over previous
import jax
import jax.numpy as jnp
from jax import lax
from jax.experimental import pallas as pl
from jax.experimental.pallas import tpu as pltpu

N_Z = 4
N_XY = 8
CHUNK = 1024
D = 1024
SL = D // N_XY
HR = CHUNK // 2


def kernel(partial, gamma):
    gamma2 = gamma.reshape(1, D)

    def body(x_ref, g_ref, out_ref, ph1_ref, y_ref,
             p1_send, p1_recv, cw_send, cw_recv, ccw_send, ccw_recv,
             xy_bar):
        my_x = lax.axis_index("x")
        my_y = lax.axis_index("y")
        my_z = lax.axis_index("z")
        zl = (my_z + N_Z - 1) % N_Z
        zr = (my_z + 1) % N_Z

        p = jnp.where(my_x == 0, my_y, 7 - my_y)

        def ring_coords(q):
            q = q % N_XY
            return q // 4, jnp.where(q < 4, q, 7 - q)

        cw_x, cw_y = ring_coords(p + 1)
        ccw_x, ccw_y = ring_coords(p + N_XY - 1)

        barrier = pltpu.get_barrier_semaphore()
        for nz in (zl, zr):
            pl.semaphore_signal(
                barrier, inc=1,
                device_id=(my_x, my_y, nz),
                device_id_type=pl.DeviceIdType.MESH,
            )
        pl.semaphore_wait(barrier, 2)

        for qx, qy in ((cw_x, cw_y), (ccw_x, ccw_y)):
            pl.semaphore_signal(
                xy_bar, inc=1,
                device_id=(qx, qy, my_z),
                device_id_type=pl.DeviceIdType.MESH,
            )

        col0 = p * SL

        ph1_rd = [[None, None] for _ in range(N_Z - 1)]

        def start_ph1(s, t):
            if s == 0:
                src = x_ref.at[0, pl.ds(zl * CHUNK + t * HR, HR), pl.ds(col0, SL)]
            else:
                src = ph1_ref.at[s - 1, pl.ds(t * HR, HR), :]
            rdma = pltpu.make_async_remote_copy(
                src_ref=src,
                dst_ref=ph1_ref.at[s, pl.ds(t * HR, HR), :],
                send_sem=p1_send.at[t, s],
                recv_sem=p1_recv.at[t, s],
                device_id=(my_x, my_y, zr),
                device_id_type=pl.DeviceIdType.MESH,
            )
            rdma.start()
            ph1_rd[s][t] = rdma

        start_ph1(0, 0)
        start_ph1(0, 1)
        for s in range(N_Z - 2):
            c = (my_z + 2 * N_Z - s - 2) % N_Z
            for t in (0, 1):
                ph1_rd[s][t].wait_recv()
                local = x_ref[0, pl.ds(c * CHUNK + t * HR, HR), pl.ds(col0, SL)]
                ph1_ref[s, pl.ds(t * HR, HR), :] = (
                    ph1_ref[s, pl.ds(t * HR, HR), :] + local
                )
                start_ph1(s + 1, t)

        pl.semaphore_wait(xy_bar, 2)

        def slice_ref(q, t):
            return y_ref.at[pl.ds(t * HR, HR), pl.ds((q % N_XY) * SL, SL)]

        def ssq_of(q):
            v = y_ref[:, pl.ds((q % N_XY) * SL, SL)]
            return jnp.sum(v * v, axis=-1, keepdims=True)

        streams = [
            ("cw", 0, 4, cw_send, cw_recv, (cw_x, cw_y)),
            ("cw", 1, 3, cw_send, cw_recv, (cw_x, cw_y)),
            ("ccw", 0, 3, ccw_send, ccw_recv, (ccw_x, ccw_y)),
            ("ccw", 1, 4, ccw_send, ccw_recv, (ccw_x, ccw_y)),
        ]
        ph2_rd = {}

        def start_ph2(si, h):
            d, t, _, ssem, rsem, (qx, qy) = streams[si]
            q = (p - h) if d == "cw" else (p + h)
            rdma = pltpu.make_async_remote_copy(
                src_ref=slice_ref(q, t),
                dst_ref=slice_ref(q, t),
                send_sem=ssem.at[t, h],
                recv_sem=rsem.at[t, h],
                device_id=(qx, qy, my_z),
                device_id_type=pl.DeviceIdType.MESH,
            )
            rdma.start()
            ph2_rd[(si, h)] = rdma

        s_last = N_Z - 2
        c = my_z
        for t, starts in ((0, (0, 2)), (1, (1, 3))):
            ph1_rd[s_last][t].wait_recv()
            local = x_ref[0, pl.ds(c * CHUNK + t * HR, HR), pl.ds(col0, SL)]
            y_ref[pl.ds(t * HR, HR), pl.ds(col0, SL)] = (
                ph1_ref[s_last, pl.ds(t * HR, HR), :] + local
            )
            for si in starts:
                start_ph2(si, 0)

        def ssq_half(q, t):
            v = y_ref[pl.ds(t * HR, HR), pl.ds((q % N_XY) * SL, SL)]
            return jnp.sum(v * v, axis=-1, keepdims=True)

        ssq = ssq_of(p)
        for h in range(3):
            if h > 0:
                ssq = ssq + ssq_of(p - h) + ssq_of(p + h)
            for si in range(4):
                nh = streams[si][2]
                if h < nh:
                    ph2_rd[(si, h)].wait_recv()
                    if h + 1 < nh:
                        start_ph2(si, h + 1)
        ssq = ssq + ssq_of(p - 3) + ssq_of(p + 3)

        inv_d = 1.0 / D
        ph2_rd[(0, 3)].wait_recv()
        ms_t = (ssq[:HR] + ssq_half(p + 4, 0)) * inv_d + 1e-6
        out_ref[pl.ds(0, HR), :] = (
            y_ref[pl.ds(0, HR), :] * lax.rsqrt(ms_t) * g_ref[0, :]
        )
        ph2_rd[(3, 3)].wait_recv()
        ms_b = (ssq[HR:] + ssq_half(p + 4, 1)) * inv_d + 1e-6
        out_ref[pl.ds(HR, HR), :] = (
            y_ref[pl.ds(HR, HR), :] * lax.rsqrt(ms_b) * g_ref[0, :]
        )

        for row in ph1_rd:
            for rdma in row:
                rdma.wait_send()
        for rdma in ph2_rd.values():
            rdma.wait_send()

    return pl.pallas_call(
        body,
        out_shape=jax.ShapeDtypeStruct((CHUNK, D), jnp.float32),
        in_specs=[
            pl.BlockSpec(memory_space=pltpu.VMEM),
            pl.BlockSpec(memory_space=pltpu.VMEM),
        ],
        out_specs=pl.BlockSpec(memory_space=pltpu.VMEM),
        scratch_shapes=[
            pltpu.VMEM((N_Z - 1, CHUNK, SL), jnp.float32),
            pltpu.VMEM((CHUNK, D), jnp.float32),
            pltpu.SemaphoreType.DMA((2, N_Z - 1)),
            pltpu.SemaphoreType.DMA((2, N_Z - 1)),
            pltpu.SemaphoreType.DMA((2, 4)),
            pltpu.SemaphoreType.DMA((2, 4)),
            pltpu.SemaphoreType.DMA((2, 4)),
            pltpu.SemaphoreType.DMA((2, 4)),
            pltpu.SemaphoreType.REGULAR,
        ],
        compiler_params=pltpu.CompilerParams(collective_id=0),
    )(partial, gamma2)
